# baseline (speedup 1.0000x reference)
import functools

import jax
import jax.numpy as jnp
from jax import lax
from jax.experimental import pallas as pl
from jax.experimental.pallas import tpu as pltpu
from jax.experimental.pallas import tpu_sc as plsc

NC = 2
NS = 16
NW = NC * NS
CH = 80
LANES = 16


def _sc_body(n_nodes, d_feat, e_per_w, nch,
             src_hbm, dst_hbm, w_hbm, feat_hbm, out_hbm,
             src_v, dst_b, w_b, rows, acc, sem_i, sem_g, sem_s):
    c = lax.axis_index("c")
    s = lax.axis_index("s")
    wid = s * NC + c
    ebase = wid * e_per_w
    nco = n_nodes // CH
    maxq = (nco + NS - 1) // NS

    pltpu.sync_copy(src_hbm.at[pl.ds(ebase, e_per_w)], src_v)

    zero = jnp.zeros((LANES,), jnp.float32)

    def zrow(i, carry):
        for cc in range(d_feat // LANES):
            rows[0, i, pl.ds(cc * LANES, LANES)] = zero
        return carry

    lax.fori_loop(0, CH, zrow, 0)

    def zchunk(q, carry):
        idx = s + q * NS

        @pl.when(idx < nco)
        def _():
            pltpu.sync_copy(rows.at[0], acc.at[pl.ds(idx * CH, CH)])

        return carry

    lax.fori_loop(0, maxq, zchunk, 0)
    plsc.subcore_barrier()

    def stage(j, p):
        off = ebase + j * CH
        pltpu.async_copy(dst_hbm.at[pl.ds(off, CH)], dst_b.at[p], sem_i.at[p])
        pltpu.async_copy(w_hbm.at[pl.ds(off, CH)], w_b.at[p], sem_i.at[p])

    def wait_stage(p):
        pltpu.make_async_copy(
            dst_hbm.at[pl.ds(0, CH)], dst_b.at[p], sem_i.at[p]).wait()
        pltpu.make_async_copy(
            w_hbm.at[pl.ds(0, CH)], w_b.at[p], sem_i.at[p]).wait()

    def gather(j, p):
        pltpu.async_copy(
            feat_hbm.at[pl.ds(0, CH)], rows.at[p],
            sem_g.at[p])

    def wait_gather(j, p):
        pltpu.make_async_copy(
            feat_hbm.at[pl.ds(0, CH)], rows.at[p],
            sem_g.at[p]).wait()

    def scatter(p2, p3):
        pltpu.async_copy(
            rows.at[p2], acc.at[dst_b.at[p3]], sem_s.at[p2], add=True)

    def wait_scatter(p2, p3):
        pltpu.make_async_copy(
            rows.at[p2], acc.at[pl.ds(0, CH)], sem_s.at[p2]).wait()

    stage(0, 0)
    wait_stage(0)
    gather(0, 0)
    stage(1, 1)

    def chunk_body(j, carry):
        p2 = lax.rem(j, 2)
        p3 = lax.rem(j, 3)
        wait_gather(j, p2)

        pltpu.async_copy(rows.at[p2], acc.at[pl.ds(0, CH)], sem_s.at[p2])

        nj = j + 1

        @pl.when(nj < nch)
        def _():
            q2 = lax.rem(nj, 2)
            q3 = lax.rem(nj, 3)
            wait_stage(q3)

            @pl.when(j >= 1)
            def _():
                wait_scatter(q2, lax.rem(j - 1, 3))

            gather(nj, q2)

        @pl.when(j + 2 < nch)
        def _():
            stage(j + 2, lax.rem(j + 2, 3))

        return carry

    lax.fori_loop(0, nch, chunk_body, 0)

    wait_scatter((nch - 2) % 2, (nch - 2) % 3)
    wait_scatter((nch - 1) % 2, (nch - 1) % 3)
    plsc.subcore_barrier()

    def dchunk(q, carry):
        idx = s + q * NS

        @pl.when(idx < nco)
        def _():
            base = idx * CH
            pltpu.sync_copy(acc.at[pl.ds(base, CH)], rows.at[0])
            pltpu.sync_copy(rows.at[0], out_hbm.at[c, pl.ds(base, CH)])

        return carry

    lax.fori_loop(0, maxq, dchunk, 0)


def _add_body(a_ref, b_ref, o_ref):
    o_ref[...] = a_ref[...] + b_ref[...]


@jax.jit
def kernel(edge_index, edge_weight, features, selfLoop):
    n_nodes, d_feat = features.shape
    n_edges = edge_weight.shape[0]
    e_per_w = n_edges // NW
    nch = e_per_w // CH

    src_flat = edge_index[1]
    dst_flat = edge_index[0]

    mesh = plsc.VectorSubcoreMesh(core_axis_name="c", subcore_axis_name="s")
    partial = pl.kernel(
        functools.partial(_sc_body, n_nodes, d_feat, e_per_w, nch),
        out_type=jax.ShapeDtypeStruct((NC, n_nodes, d_feat), jnp.float32),
        mesh=mesh,
        compiler_params=pltpu.CompilerParams(needs_layout_passes=False),
        scratch_types=[
            pltpu.VMEM((e_per_w,), jnp.int32),
            pltpu.VMEM((3, CH), jnp.int32),
            pltpu.VMEM((3, CH), jnp.float32),
            pltpu.VMEM((2, CH, d_feat), jnp.float32),
            pltpu.VMEM_SHARED((n_nodes, d_feat), jnp.float32),
            pltpu.SemaphoreType.DMA((3,)),
            pltpu.SemaphoreType.DMA((2,)),
            pltpu.SemaphoreType.DMA((2,)),
        ],
    )(src_flat, dst_flat, edge_weight, features)

    blk = 1000
    out = pl.pallas_call(
        _add_body,
        out_shape=jax.ShapeDtypeStruct((n_nodes, d_feat), jnp.float32),
        grid=(n_nodes // blk,),
        in_specs=[
            pl.BlockSpec((blk, d_feat), lambda i: (i, 0)),
            pl.BlockSpec((blk, d_feat), lambda i: (i, 0)),
        ],
        out_specs=pl.BlockSpec((blk, d_feat), lambda i: (i, 0)),
    )(partial[0], partial[1])
    return out


# speedup vs baseline: 2.1493x; 2.1493x over previous
import functools

import jax
import jax.numpy as jnp
from jax import lax
from jax.experimental import pallas as pl
from jax.experimental.pallas import tpu as pltpu
from jax.experimental.pallas import tpu_sc as plsc

NC = 2
NS = 16
NW = NC * NS
CH = 80
RI = 5
RB = 3
LANES = 16


def _sc_body(n_nodes, d_feat, e_per_w, nch,
             src_hbm, dst_hbm, w_hbm, feat_hbm, out_hbm,
             src_b, dst_b, w_b, rows, acc, sem_i, sem_g, sem_s):
    c = lax.axis_index("c")
    s = lax.axis_index("s")
    wid = s * NC + c
    ebase = wid * e_per_w
    nco = n_nodes // CH
    maxq = (nco + NS - 1) // NS

    zero = jnp.zeros((LANES,), jnp.float32)

    def zrow(i, carry):
        for cc in range(d_feat // LANES):
            rows[0, i, pl.ds(cc * LANES, LANES)] = zero
        return carry

    lax.fori_loop(0, CH, zrow, 0)

    def zchunk(q, carry):
        idx = s + q * NS

        @pl.when(idx < nco)
        def _():
            pltpu.sync_copy(rows.at[0], acc.at[pl.ds(idx * CH, CH)])

        return carry

    lax.fori_loop(0, maxq, zchunk, 0)
    plsc.subcore_barrier()

    def stage(j, p):
        off = ebase + j * CH
        pltpu.async_copy(src_hbm.at[pl.ds(off, CH)], src_b.at[p], sem_i.at[p])
        pltpu.async_copy(dst_hbm.at[pl.ds(off, CH)], dst_b.at[p], sem_i.at[p])
        pltpu.async_copy(w_hbm.at[pl.ds(off, CH)], w_b.at[p], sem_i.at[p])

    def wait_stage(p):
        pltpu.make_async_copy(
            src_hbm.at[pl.ds(0, CH)], src_b.at[p], sem_i.at[p]).wait()
        pltpu.make_async_copy(
            dst_hbm.at[pl.ds(0, CH)], dst_b.at[p], sem_i.at[p]).wait()
        pltpu.make_async_copy(
            w_hbm.at[pl.ds(0, CH)], w_b.at[p], sem_i.at[p]).wait()

    def gather(pi, pb):
        pltpu.async_copy(
            feat_hbm.at[src_b.at[pi]], rows.at[pb], sem_g.at[pb])

    def wait_gather(pi, pb):
        pltpu.make_async_copy(
            feat_hbm.at[src_b.at[pi]], rows.at[pb], sem_g.at[pb]).wait()

    def scatter(pb, pi):
        pltpu.async_copy(
            rows.at[pb], acc.at[dst_b.at[pi]], sem_s.at[pb], add=True)

    def wait_scatter(pb):
        pltpu.make_async_copy(
            rows.at[pb], acc.at[dst_b.at[0]], sem_s.at[pb]).wait()

    stage(0, 0)
    wait_stage(0)
    gather(0, 0)
    stage(1, 1)
    stage(2, 2)

    def chunk_body(j, carry):
        p3 = lax.rem(j, RB)
        p5 = lax.rem(j, RI)

        @pl.when(j >= 2)
        def _():
            wait_scatter(lax.rem(j - 2, RB))

        nj = j + 1

        @pl.when(nj < nch)
        def _():
            q5 = lax.rem(nj, RI)
            wait_stage(q5)
            gather(q5, lax.rem(nj, RB))

        @pl.when(j + 3 < nch)
        def _():
            stage(j + 3, lax.rem(j + 3, RI))

        wait_gather(p5, p3)

        p5v = jnp.full((LANES,), p5, jnp.int32)

        @plsc.parallel_loop(0, CH, unroll=4)
        def _(i):
            wsplat = plsc.load_gather(
                w_b, [p5v, jnp.full((LANES,), i, jnp.int32)])
            for cc in range(d_feat // LANES):
                sl = pl.ds(cc * LANES, LANES)
                rows[p3, i, sl] = rows[p3, i, sl] * wsplat

        scatter(p3, p5)
        return carry

    lax.fori_loop(0, nch, chunk_body, 0)

    wait_scatter((nch - 2) % RB)
    wait_scatter((nch - 1) % RB)
    plsc.subcore_barrier()

    def dchunk(q, carry):
        idx = s + q * NS

        @pl.when(idx < nco)
        def _():
            base = idx * CH
            pltpu.sync_copy(acc.at[pl.ds(base, CH)], rows.at[0])
            pltpu.sync_copy(rows.at[0], out_hbm.at[c, pl.ds(base, CH)])

        return carry

    lax.fori_loop(0, maxq, dchunk, 0)


def _add_body(a_ref, b_ref, o_ref):
    o_ref[...] = a_ref[...] + b_ref[...]


@jax.jit
def kernel(edge_index, edge_weight, features, selfLoop):
    n_nodes, d_feat = features.shape
    n_edges = edge_weight.shape[0]
    e_per_w = n_edges // NW
    nch = e_per_w // CH

    src_flat = edge_index[1]
    dst_flat = edge_index[0]

    mesh = plsc.VectorSubcoreMesh(core_axis_name="c", subcore_axis_name="s")
    partial = pl.kernel(
        functools.partial(_sc_body, n_nodes, d_feat, e_per_w, nch),
        out_type=jax.ShapeDtypeStruct((NC, n_nodes, d_feat), jnp.float32),
        mesh=mesh,
        compiler_params=pltpu.CompilerParams(needs_layout_passes=False),
        scratch_types=[
            pltpu.VMEM((RI, CH), jnp.int32),
            pltpu.VMEM((RI, CH), jnp.int32),
            pltpu.VMEM((RI, CH), jnp.float32),
            pltpu.VMEM((RB, CH, d_feat), jnp.float32),
            pltpu.VMEM_SHARED((n_nodes, d_feat), jnp.float32),
            pltpu.SemaphoreType.DMA((RI,)),
            pltpu.SemaphoreType.DMA((RB,)),
            pltpu.SemaphoreType.DMA((RB,)),
        ],
    )(src_flat, dst_flat, edge_weight, features)

    blk = 1000
    out = pl.pallas_call(
        _add_body,
        out_shape=jax.ShapeDtypeStruct((n_nodes, d_feat), jnp.float32),
        grid=(n_nodes // blk,),
        in_specs=[
            pl.BlockSpec((blk, d_feat), lambda i: (i, 0)),
            pl.BlockSpec((blk, d_feat), lambda i: (i, 0)),
        ],
        out_specs=pl.BlockSpec((blk, d_feat), lambda i: (i, 0)),
    )(partial[0], partial[1])
    return out
